# Initial kernel scaffold; baseline (speedup 1.0000x reference)
import jax
import jax.numpy as jnp
from jax.experimental import pallas as pl


def kernel(x, edge_index, W1, b1, W2, b2, Wc, bc):
    raise NotImplementedError("write your pallas kernel here")


# speedup vs baseline: 25.3324x; 25.3324x over previous
import functools

import jax
import jax.numpy as jnp
from jax import lax
from jax.experimental import pallas as pl
from jax.experimental.pallas import tpu as pltpu
from jax.experimental.pallas import tpu_sc as plsc

N = 10000
NPAD = 10240
E = 320000
H = 32
F = 128
O = 4

NC = 2
NS = 16
NW = NC * NS
CHUNK = 128
CH = 79
EPAD = NW * CH * CHUNK
ROWS_PER_TILE = NPAD // NS
RB = 512


def _sc_mesh():
    return plsc.VectorSubcoreMesh(core_axis_name="c", subcore_axis_name="s")


@jax.jit
def _sc_degree(dst3):
    @functools.partial(
        pl.kernel,
        mesh=_sc_mesh(),
        out_type=jax.ShapeDtypeStruct((NC, NPAD), jnp.float32),
        scratch_types=[
            pltpu.VMEM((CH, CHUNK), jnp.int32),
            pltpu.VMEM((CHUNK,), jnp.float32),
            pltpu.VMEM((CHUNK,), jnp.float32),
            pltpu.VMEM_SHARED((NPAD,), jnp.float32),
        ],
        compiler_params=pltpu.CompilerParams(use_tc_tiling_on_sc=False),
    )
    def k(dst_hbm, out_hbm, dst_v, ones_v, zeros_v, deg_sh):
        c = lax.axis_index("c")
        s = lax.axis_index("s")
        wid = c * NS + s

        def init(i, _):
            ones_v[pl.ds(i * 16, 16)] = jnp.ones((16,), jnp.float32)
            zeros_v[pl.ds(i * 16, 16)] = jnp.zeros((16,), jnp.float32)
            return 0

        lax.fori_loop(0, CHUNK // 16, init, 0)

        def zblk(i, _):
            pltpu.sync_copy(
                zeros_v, deg_sh.at[pl.ds(s * ROWS_PER_TILE + i * CHUNK, CHUNK)])
            return 0

        lax.fori_loop(0, ROWS_PER_TILE // CHUNK, zblk, 0)
        pltpu.sync_copy(dst_hbm.at[wid], dst_v)
        plsc.subcore_barrier()

        def body(j, _):
            pltpu.sync_copy(ones_v, deg_sh.at[dst_v.at[j]], add=True)
            return 0

        lax.fori_loop(0, CH, body, 0)
        plsc.subcore_barrier()
        pltpu.sync_copy(deg_sh.at[pl.ds(s * ROWS_PER_TILE, ROWS_PER_TILE)],
                        out_hbm.at[c, pl.ds(s * ROWS_PER_TILE, ROWS_PER_TILE)])

    return k(dst3)


@jax.jit
def _sc_scatter(y, src3, dst3):
    @functools.partial(
        pl.kernel,
        mesh=_sc_mesh(),
        out_type=jax.ShapeDtypeStruct((NC, NPAD, H), jnp.float32),
        scratch_types=[
            pltpu.VMEM((CH, CHUNK), jnp.int32),
            pltpu.VMEM((CH, CHUNK), jnp.int32),
            pltpu.VMEM((CHUNK, H), jnp.float32),
            pltpu.VMEM((CHUNK, H), jnp.float32),
            pltpu.VMEM_SHARED((NPAD, H), jnp.float32),
            pltpu.SemaphoreType.DMA,
        ],
        compiler_params=pltpu.CompilerParams(use_tc_tiling_on_sc=False),
    )
    def k(y_hbm, src_hbm, dst_hbm, out_hbm, src_v, dst_v, rows_v, zeros_v,
          acc_sh, sem):
        c = lax.axis_index("c")
        s = lax.axis_index("s")
        wid = c * NS + s

        def zrow(i, _):
            zeros_v[i, pl.ds(0, 16)] = jnp.zeros((16,), jnp.float32)
            zeros_v[i, pl.ds(16, 16)] = jnp.zeros((16,), jnp.float32)
            return 0

        lax.fori_loop(0, CHUNK, zrow, 0)

        def zblk(i, _):
            pltpu.sync_copy(
                zeros_v, acc_sh.at[pl.ds(s * ROWS_PER_TILE + i * CHUNK, CHUNK)])
            return 0

        lax.fori_loop(0, ROWS_PER_TILE // CHUNK, zblk, 0)
        pltpu.sync_copy(src_hbm.at[wid], src_v)
        pltpu.sync_copy(dst_hbm.at[wid], dst_v)
        plsc.subcore_barrier()

        def body(j, _):
            pltpu.async_copy(y_hbm.at[src_v.at[j]], rows_v, sem).wait()
            pltpu.sync_copy(rows_v, acc_sh.at[dst_v.at[j]], add=True)
            return 0

        lax.fori_loop(0, CH, body, 0)
        plsc.subcore_barrier()
        pltpu.sync_copy(acc_sh.at[pl.ds(s * ROWS_PER_TILE, ROWS_PER_TILE)],
                        out_hbm.at[c, pl.ds(s * ROWS_PER_TILE, ROWS_PER_TILE)])

    return k(y, src3, dst3)


def _tc1(x_pad, W1, degA, degB):
    def body(x_ref, w_ref, da_ref, db_ref, y_ref, s_ref, dis_ref):
        dis = lax.rsqrt(da_ref[...] + db_ref[...] + 1.0)
        xw = jnp.dot(x_ref[...], w_ref[...], preferred_element_type=jnp.float32)
        y = xw * dis
        y_ref[...] = y
        s_ref[...] = y * dis
        dis_ref[...] = dis

    g = NPAD // RB
    return pl.pallas_call(
        body,
        grid=(g,),
        in_specs=[
            pl.BlockSpec((RB, F), lambda i: (i, 0)),
            pl.BlockSpec((F, H), lambda i: (0, 0)),
            pl.BlockSpec((RB, 1), lambda i: (i, 0)),
            pl.BlockSpec((RB, 1), lambda i: (i, 0)),
        ],
        out_specs=[
            pl.BlockSpec((RB, H), lambda i: (i, 0)),
            pl.BlockSpec((RB, H), lambda i: (i, 0)),
            pl.BlockSpec((RB, 1), lambda i: (i, 0)),
        ],
        out_shape=[
            jax.ShapeDtypeStruct((NPAD, H), jnp.float32),
            jax.ShapeDtypeStruct((NPAD, H), jnp.float32),
            jax.ShapeDtypeStruct((NPAD, 1), jnp.float32),
        ],
    )(x_pad, W1, degA, degB)


def _tc2(accA, accB, s1, dis, b1, W2):
    def body(aa_ref, ab_ref, s1_ref, dis_ref, b_ref, w_ref, y_ref, s_ref):
        dis = dis_ref[...]
        h = jnp.maximum(
            (aa_ref[...] + ab_ref[...]) * dis + s1_ref[...] + b_ref[...], 0.0)
        xw = jnp.dot(h, w_ref[...], preferred_element_type=jnp.float32)
        y = xw * dis
        y_ref[...] = y
        s_ref[...] = y * dis

    g = NPAD // RB
    return pl.pallas_call(
        body,
        grid=(g,),
        in_specs=[
            pl.BlockSpec((RB, H), lambda i: (i, 0)),
            pl.BlockSpec((RB, H), lambda i: (i, 0)),
            pl.BlockSpec((RB, H), lambda i: (i, 0)),
            pl.BlockSpec((RB, 1), lambda i: (i, 0)),
            pl.BlockSpec((1, H), lambda i: (0, 0)),
            pl.BlockSpec((H, H), lambda i: (0, 0)),
        ],
        out_specs=[
            pl.BlockSpec((RB, H), lambda i: (i, 0)),
            pl.BlockSpec((RB, H), lambda i: (i, 0)),
        ],
        out_shape=[
            jax.ShapeDtypeStruct((NPAD, H), jnp.float32),
            jax.ShapeDtypeStruct((NPAD, H), jnp.float32),
        ],
    )(accA, accB, s1, dis, b1, W2)


def _tc3(accA, accB, s2, dis, b2, Wc, bc):
    def body(aa_ref, ab_ref, s2_ref, dis_ref, b_ref, wc_ref, bc_ref, o_ref):
        dis = dis_ref[...]
        h = jnp.maximum(
            (aa_ref[...] + ab_ref[...]) * dis + s2_ref[...] + b_ref[...], 0.0)
        o_ref[...] = jnp.dot(
            h, wc_ref[...], preferred_element_type=jnp.float32) + bc_ref[...]

    g = NPAD // RB
    return pl.pallas_call(
        body,
        grid=(g,),
        in_specs=[
            pl.BlockSpec((RB, H), lambda i: (i, 0)),
            pl.BlockSpec((RB, H), lambda i: (i, 0)),
            pl.BlockSpec((RB, H), lambda i: (i, 0)),
            pl.BlockSpec((RB, 1), lambda i: (i, 0)),
            pl.BlockSpec((1, H), lambda i: (0, 0)),
            pl.BlockSpec((H, O), lambda i: (0, 0)),
            pl.BlockSpec((1, O), lambda i: (0, 0)),
        ],
        out_specs=pl.BlockSpec((RB, O), lambda i: (i, 0)),
        out_shape=jax.ShapeDtypeStruct((NPAD, O), jnp.float32),
    )(accA, accB, s2, dis, b2, Wc, bc)


def kernel(x, edge_index, W1, b1, W2, b2, Wc, bc):
    ei = edge_index.astype(jnp.int32)
    pad = jnp.full((EPAD - E,), N, jnp.int32)
    src3 = jnp.concatenate([ei[0], pad]).reshape(NW, CH, CHUNK)
    dst3 = jnp.concatenate([ei[1], pad]).reshape(NW, CH, CHUNK)
    x_pad = jnp.pad(x, ((0, NPAD - N), (0, 0)))

    degp = _sc_degree(dst3)
    y1, s1, dis = _tc1(x_pad, W1,
                       degp[0].reshape(NPAD, 1), degp[1].reshape(NPAD, 1))
    acc1 = _sc_scatter(y1, src3, dst3)
    y2, s2 = _tc2(acc1[0], acc1[1], s1, dis,
                  b1.reshape(1, H), W2)
    acc2 = _sc_scatter(y2, src3, dst3)
    out = _tc3(acc2[0], acc2[1], s2, dis,
               b2.reshape(1, H), Wc, bc.reshape(1, O))
    return out[:N]


# speedup vs baseline: 32.6068x; 1.2872x over previous
import functools

import jax
import jax.numpy as jnp
from jax import lax
from jax.experimental import pallas as pl
from jax.experimental.pallas import tpu as pltpu
from jax.experimental.pallas import tpu_sc as plsc

N = 10000
NPAD = 10240
E = 320000
H = 32
F = 128
O = 4

NC = 2
NS = 16
NW = NC * NS
CHUNK = 128
CH = 79
EPAD = NW * CH * CHUNK
ROWS_PER_TILE = NPAD // NS
RB = 512


def _sc_mesh():
    return plsc.VectorSubcoreMesh(core_axis_name="c", subcore_axis_name="s")


@jax.jit
def _sc_degree(dst3):
    @functools.partial(
        pl.kernel,
        mesh=_sc_mesh(),
        out_type=jax.ShapeDtypeStruct((NC, NPAD), jnp.float32),
        scratch_types=[
            pltpu.VMEM((CH, CHUNK), jnp.int32),
            pltpu.VMEM((CHUNK,), jnp.float32),
            pltpu.VMEM((CHUNK,), jnp.float32),
            pltpu.VMEM_SHARED((NPAD,), jnp.float32),
        ],
        compiler_params=pltpu.CompilerParams(use_tc_tiling_on_sc=False),
    )
    def k(dst_hbm, out_hbm, dst_v, ones_v, zeros_v, deg_sh):
        c = lax.axis_index("c")
        s = lax.axis_index("s")
        wid = c * NS + s

        def init(i, _):
            ones_v[pl.ds(i * 16, 16)] = jnp.ones((16,), jnp.float32)
            zeros_v[pl.ds(i * 16, 16)] = jnp.zeros((16,), jnp.float32)
            return 0

        lax.fori_loop(0, CHUNK // 16, init, 0)

        def zblk(i, _):
            pltpu.sync_copy(
                zeros_v, deg_sh.at[pl.ds(s * ROWS_PER_TILE + i * CHUNK, CHUNK)])
            return 0

        lax.fori_loop(0, ROWS_PER_TILE // CHUNK, zblk, 0)
        pltpu.sync_copy(dst_hbm.at[wid], dst_v)
        plsc.subcore_barrier()

        def body(j, _):
            pltpu.sync_copy(ones_v, deg_sh.at[dst_v.at[j]], add=True)
            return 0

        lax.fori_loop(0, CH, body, 0)
        plsc.subcore_barrier()
        pltpu.sync_copy(deg_sh.at[pl.ds(s * ROWS_PER_TILE, ROWS_PER_TILE)],
                        out_hbm.at[c, pl.ds(s * ROWS_PER_TILE, ROWS_PER_TILE)])

    return k(dst3)


@jax.jit
def _sc_scatter(y, src3, dst3):
    @functools.partial(
        pl.kernel,
        mesh=_sc_mesh(),
        out_type=jax.ShapeDtypeStruct((NC, NPAD, H), jnp.float32),
        scratch_types=[
            pltpu.VMEM((CH, CHUNK), jnp.int32),
            pltpu.VMEM((CH, CHUNK), jnp.int32),
            pltpu.VMEM((2, CHUNK, H), jnp.float32),
            pltpu.VMEM((CHUNK, H), jnp.float32),
            pltpu.VMEM_SHARED((NPAD, H), jnp.float32),
            pltpu.SemaphoreType.DMA,
            pltpu.SemaphoreType.DMA,
        ],
        compiler_params=pltpu.CompilerParams(use_tc_tiling_on_sc=False),
    )
    def k(y_hbm, src_hbm, dst_hbm, out_hbm, src_v, dst_v, rows_v, zeros_v,
          acc_sh, sem_g, sem_s):
        c = lax.axis_index("c")
        s = lax.axis_index("s")
        wid = c * NS + s

        def zrow(i, _):
            zeros_v[i, pl.ds(0, 16)] = jnp.zeros((16,), jnp.float32)
            zeros_v[i, pl.ds(16, 16)] = jnp.zeros((16,), jnp.float32)
            return 0

        lax.fori_loop(0, CHUNK, zrow, 0)

        def zblk(i, _):
            pltpu.sync_copy(
                zeros_v, acc_sh.at[pl.ds(s * ROWS_PER_TILE + i * CHUNK, CHUNK)])
            return 0

        lax.fori_loop(0, ROWS_PER_TILE // CHUNK, zblk, 0)
        pltpu.sync_copy(src_hbm.at[wid], src_v)
        pltpu.sync_copy(dst_hbm.at[wid], dst_v)
        plsc.subcore_barrier()

        pltpu.async_copy(y_hbm.at[src_v.at[0]], rows_v.at[0], sem_g)

        def body(j, _):
            p = j % 2
            pltpu.make_async_copy(
                y_hbm.at[src_v.at[j]], rows_v.at[p], sem_g).wait()
            pltpu.async_copy(
                rows_v.at[p], acc_sh.at[dst_v.at[j]], sem_s, add=True)

            @pl.when(j >= 1)
            def _():
                pltpu.make_async_copy(
                    rows_v.at[1 - p], acc_sh.at[dst_v.at[j]], sem_s).wait()

            @pl.when(j < CH - 1)
            def _():
                pltpu.async_copy(
                    y_hbm.at[src_v.at[j + 1]], rows_v.at[1 - p], sem_g)

            return 0

        lax.fori_loop(0, CH, body, 0)
        pltpu.make_async_copy(
            rows_v.at[(CH - 1) % 2], acc_sh.at[dst_v.at[0]], sem_s).wait()
        plsc.subcore_barrier()
        pltpu.sync_copy(acc_sh.at[pl.ds(s * ROWS_PER_TILE, ROWS_PER_TILE)],
                        out_hbm.at[c, pl.ds(s * ROWS_PER_TILE, ROWS_PER_TILE)])

    return k(y, src3, dst3)


def _tc1(x_pad, W1, degA, degB):
    def body(x_ref, w_ref, da_ref, db_ref, y_ref, s_ref, dis_ref):
        dis = lax.rsqrt(da_ref[...] + db_ref[...] + 1.0)
        xw = jnp.dot(x_ref[...], w_ref[...], preferred_element_type=jnp.float32)
        y = xw * dis
        y_ref[...] = y
        s_ref[...] = y * dis
        dis_ref[...] = dis

    g = NPAD // RB
    return pl.pallas_call(
        body,
        grid=(g,),
        in_specs=[
            pl.BlockSpec((RB, F), lambda i: (i, 0)),
            pl.BlockSpec((F, H), lambda i: (0, 0)),
            pl.BlockSpec((RB, 1), lambda i: (i, 0)),
            pl.BlockSpec((RB, 1), lambda i: (i, 0)),
        ],
        out_specs=[
            pl.BlockSpec((RB, H), lambda i: (i, 0)),
            pl.BlockSpec((RB, H), lambda i: (i, 0)),
            pl.BlockSpec((RB, 1), lambda i: (i, 0)),
        ],
        out_shape=[
            jax.ShapeDtypeStruct((NPAD, H), jnp.float32),
            jax.ShapeDtypeStruct((NPAD, H), jnp.float32),
            jax.ShapeDtypeStruct((NPAD, 1), jnp.float32),
        ],
    )(x_pad, W1, degA, degB)


def _tc2(accA, accB, s1, dis, b1, W2):
    def body(aa_ref, ab_ref, s1_ref, dis_ref, b_ref, w_ref, y_ref, s_ref):
        dis = dis_ref[...]
        h = jnp.maximum(
            (aa_ref[...] + ab_ref[...]) * dis + s1_ref[...] + b_ref[...], 0.0)
        xw = jnp.dot(h, w_ref[...], preferred_element_type=jnp.float32)
        y = xw * dis
        y_ref[...] = y
        s_ref[...] = y * dis

    g = NPAD // RB
    return pl.pallas_call(
        body,
        grid=(g,),
        in_specs=[
            pl.BlockSpec((RB, H), lambda i: (i, 0)),
            pl.BlockSpec((RB, H), lambda i: (i, 0)),
            pl.BlockSpec((RB, H), lambda i: (i, 0)),
            pl.BlockSpec((RB, 1), lambda i: (i, 0)),
            pl.BlockSpec((1, H), lambda i: (0, 0)),
            pl.BlockSpec((H, H), lambda i: (0, 0)),
        ],
        out_specs=[
            pl.BlockSpec((RB, H), lambda i: (i, 0)),
            pl.BlockSpec((RB, H), lambda i: (i, 0)),
        ],
        out_shape=[
            jax.ShapeDtypeStruct((NPAD, H), jnp.float32),
            jax.ShapeDtypeStruct((NPAD, H), jnp.float32),
        ],
    )(accA, accB, s1, dis, b1, W2)


def _tc3(accA, accB, s2, dis, b2, Wc, bc):
    def body(aa_ref, ab_ref, s2_ref, dis_ref, b_ref, wc_ref, bc_ref, o_ref):
        dis = dis_ref[...]
        h = jnp.maximum(
            (aa_ref[...] + ab_ref[...]) * dis + s2_ref[...] + b_ref[...], 0.0)
        o_ref[...] = jnp.dot(
            h, wc_ref[...], preferred_element_type=jnp.float32) + bc_ref[...]

    g = NPAD // RB
    return pl.pallas_call(
        body,
        grid=(g,),
        in_specs=[
            pl.BlockSpec((RB, H), lambda i: (i, 0)),
            pl.BlockSpec((RB, H), lambda i: (i, 0)),
            pl.BlockSpec((RB, H), lambda i: (i, 0)),
            pl.BlockSpec((RB, 1), lambda i: (i, 0)),
            pl.BlockSpec((1, H), lambda i: (0, 0)),
            pl.BlockSpec((H, O), lambda i: (0, 0)),
            pl.BlockSpec((1, O), lambda i: (0, 0)),
        ],
        out_specs=pl.BlockSpec((RB, O), lambda i: (i, 0)),
        out_shape=jax.ShapeDtypeStruct((NPAD, O), jnp.float32),
    )(accA, accB, s2, dis, b2, Wc, bc)


def kernel(x, edge_index, W1, b1, W2, b2, Wc, bc):
    ei = edge_index.astype(jnp.int32)
    ar = jnp.arange(EPAD - E, dtype=jnp.int32)
    src3 = jnp.concatenate([ei[0], ar % NPAD]).reshape(NW, CH, CHUNK)
    dst3 = jnp.concatenate([ei[1], N + ar % (NPAD - N)]).reshape(NW, CH, CHUNK)
    x_pad = jnp.pad(x, ((0, NPAD - N), (0, 0)))

    degp = _sc_degree(dst3)
    y1, s1, dis = _tc1(x_pad, W1,
                       degp[0].reshape(NPAD, 1), degp[1].reshape(NPAD, 1))
    acc1 = _sc_scatter(y1, src3, dst3)
    y2, s2 = _tc2(acc1[0], acc1[1], s1, dis,
                  b1.reshape(1, H), W2)
    acc2 = _sc_scatter(y2, src3, dst3)
    out = _tc3(acc2[0], acc2[1], s2, dis,
               b2.reshape(1, H), Wc, bc.reshape(1, O))
    return out[:N]


# speedup vs baseline: 43.5872x; 1.3368x over previous
import functools

import jax
import jax.numpy as jnp
from jax import lax
from jax.experimental import pallas as pl
from jax.experimental.pallas import tpu as pltpu
from jax.experimental.pallas import tpu_sc as plsc

N = 10000
NPAD = 10240
E = 320000
H = 32
F = 128
O = 4

NC = 2
NS = 16
NW = NC * NS
CHUNK = 128
CH = 79
EPAD = NW * CH * CHUNK
ROWS_PER_TILE = NPAD // NS
RB = 512


def _sc_mesh():
    return plsc.VectorSubcoreMesh(core_axis_name="c", subcore_axis_name="s")


@jax.jit
def _sc_degree(dst3):
    @functools.partial(
        pl.kernel,
        mesh=_sc_mesh(),
        out_type=jax.ShapeDtypeStruct((NC, NPAD), jnp.float32),
        scratch_types=[
            pltpu.VMEM((CH, CHUNK), jnp.int32),
            pltpu.VMEM((CHUNK,), jnp.float32),
            pltpu.VMEM((CHUNK,), jnp.float32),
            pltpu.VMEM_SHARED((NPAD,), jnp.float32),
        ],
        compiler_params=pltpu.CompilerParams(use_tc_tiling_on_sc=False),
    )
    def k(dst_hbm, out_hbm, dst_v, ones_v, zeros_v, deg_sh):
        c = lax.axis_index("c")
        s = lax.axis_index("s")
        wid = c * NS + s

        def init(i, _):
            ones_v[pl.ds(i * 16, 16)] = jnp.ones((16,), jnp.float32)
            zeros_v[pl.ds(i * 16, 16)] = jnp.zeros((16,), jnp.float32)
            return 0

        lax.fori_loop(0, CHUNK // 16, init, 0)

        def zblk(i, _):
            pltpu.sync_copy(
                zeros_v, deg_sh.at[pl.ds(s * ROWS_PER_TILE + i * CHUNK, CHUNK)])
            return 0

        lax.fori_loop(0, ROWS_PER_TILE // CHUNK, zblk, 0)
        pltpu.sync_copy(dst_hbm.at[wid], dst_v)
        plsc.subcore_barrier()

        def body(j, _):
            pltpu.sync_copy(ones_v, deg_sh.at[dst_v.at[j]], add=True)
            return 0

        lax.fori_loop(0, CH, body, 0)
        plsc.subcore_barrier()
        pltpu.sync_copy(deg_sh.at[pl.ds(s * ROWS_PER_TILE, ROWS_PER_TILE)],
                        out_hbm.at[c, pl.ds(s * ROWS_PER_TILE, ROWS_PER_TILE)])

    return k(dst3)


@jax.jit
def _sc_scatter(y, src3, dst3):
    @functools.partial(
        pl.kernel,
        mesh=_sc_mesh(),
        out_type=jax.ShapeDtypeStruct((NC, NPAD, H), jnp.float32),
        scratch_types=[
            pltpu.VMEM((CH, CHUNK), jnp.int32),
            pltpu.VMEM((CH, CHUNK), jnp.int32),
            pltpu.VMEM((4, CHUNK, H), jnp.float32),
            pltpu.VMEM((CHUNK, H), jnp.float32),
            pltpu.VMEM_SHARED((NPAD, H), jnp.float32),
            pltpu.SemaphoreType.DMA,
            pltpu.SemaphoreType.DMA,
        ],
        compiler_params=pltpu.CompilerParams(use_tc_tiling_on_sc=False),
    )
    def k(y_hbm, src_hbm, dst_hbm, out_hbm, src_v, dst_v, rows_v, zeros_v,
          acc_sh, sem_g, sem_s):
        c = lax.axis_index("c")
        s = lax.axis_index("s")
        wid = c * NS + s

        def zrow(i, _):
            zeros_v[i, pl.ds(0, 16)] = jnp.zeros((16,), jnp.float32)
            zeros_v[i, pl.ds(16, 16)] = jnp.zeros((16,), jnp.float32)
            return 0

        lax.fori_loop(0, CHUNK, zrow, 0)

        def zblk(i, _):
            pltpu.sync_copy(
                zeros_v, acc_sh.at[pl.ds(s * ROWS_PER_TILE + i * CHUNK, CHUNK)])
            return 0

        lax.fori_loop(0, ROWS_PER_TILE // CHUNK, zblk, 0)
        pltpu.sync_copy(src_hbm.at[wid], src_v)
        pltpu.sync_copy(dst_hbm.at[wid], dst_v)
        plsc.subcore_barrier()

        pltpu.async_copy(y_hbm.at[src_v.at[0]], rows_v.at[0], sem_g)
        pltpu.async_copy(y_hbm.at[src_v.at[1]], rows_v.at[1], sem_g)
        pltpu.async_copy(y_hbm.at[src_v.at[2]], rows_v.at[2], sem_g)

        def body(j, _):
            p = j % 4
            pltpu.make_async_copy(
                y_hbm.at[src_v.at[j]], rows_v.at[p], sem_g).wait()
            pltpu.async_copy(
                rows_v.at[p], acc_sh.at[dst_v.at[j]], sem_s, add=True)

            @pl.when(j >= 1)
            def _():
                pltpu.make_async_copy(
                    rows_v.at[p], acc_sh.at[dst_v.at[j]], sem_s).wait()

            @pl.when(j < CH - 3)
            def _():
                pltpu.async_copy(
                    y_hbm.at[src_v.at[j + 3]], rows_v.at[(j + 3) % 4], sem_g)

            return 0

        lax.fori_loop(0, CH, body, 0)
        pltpu.make_async_copy(
            rows_v.at[(CH - 1) % 4], acc_sh.at[dst_v.at[0]], sem_s).wait()
        plsc.subcore_barrier()
        pltpu.sync_copy(acc_sh.at[pl.ds(s * ROWS_PER_TILE, ROWS_PER_TILE)],
                        out_hbm.at[c, pl.ds(s * ROWS_PER_TILE, ROWS_PER_TILE)])

    return k(y, src3, dst3)


def _tc1(x_pad, W1, degA, degB):
    def body(x_ref, w_ref, da_ref, db_ref, y_ref, s_ref, dis_ref):
        dis = lax.rsqrt(da_ref[...] + db_ref[...] + 1.0)
        xw = jnp.dot(x_ref[...], w_ref[...], preferred_element_type=jnp.float32)
        y = xw * dis
        y_ref[...] = y
        s_ref[...] = y * dis
        dis_ref[...] = dis

    g = NPAD // RB
    return pl.pallas_call(
        body,
        grid=(g,),
        in_specs=[
            pl.BlockSpec((RB, F), lambda i: (i, 0)),
            pl.BlockSpec((F, H), lambda i: (0, 0)),
            pl.BlockSpec((RB, 1), lambda i: (i, 0)),
            pl.BlockSpec((RB, 1), lambda i: (i, 0)),
        ],
        out_specs=[
            pl.BlockSpec((RB, H), lambda i: (i, 0)),
            pl.BlockSpec((RB, H), lambda i: (i, 0)),
            pl.BlockSpec((RB, 1), lambda i: (i, 0)),
        ],
        out_shape=[
            jax.ShapeDtypeStruct((NPAD, H), jnp.float32),
            jax.ShapeDtypeStruct((NPAD, H), jnp.float32),
            jax.ShapeDtypeStruct((NPAD, 1), jnp.float32),
        ],
    )(x_pad, W1, degA, degB)


def _tc2(accA, accB, s1, dis, b1, W2):
    def body(aa_ref, ab_ref, s1_ref, dis_ref, b_ref, w_ref, y_ref, s_ref):
        dis = dis_ref[...]
        h = jnp.maximum(
            (aa_ref[...] + ab_ref[...]) * dis + s1_ref[...] + b_ref[...], 0.0)
        xw = jnp.dot(h, w_ref[...], preferred_element_type=jnp.float32)
        y = xw * dis
        y_ref[...] = y
        s_ref[...] = y * dis

    g = NPAD // RB
    return pl.pallas_call(
        body,
        grid=(g,),
        in_specs=[
            pl.BlockSpec((RB, H), lambda i: (i, 0)),
            pl.BlockSpec((RB, H), lambda i: (i, 0)),
            pl.BlockSpec((RB, H), lambda i: (i, 0)),
            pl.BlockSpec((RB, 1), lambda i: (i, 0)),
            pl.BlockSpec((1, H), lambda i: (0, 0)),
            pl.BlockSpec((H, H), lambda i: (0, 0)),
        ],
        out_specs=[
            pl.BlockSpec((RB, H), lambda i: (i, 0)),
            pl.BlockSpec((RB, H), lambda i: (i, 0)),
        ],
        out_shape=[
            jax.ShapeDtypeStruct((NPAD, H), jnp.float32),
            jax.ShapeDtypeStruct((NPAD, H), jnp.float32),
        ],
    )(accA, accB, s1, dis, b1, W2)


def _tc3(accA, accB, s2, dis, b2, Wc, bc):
    def body(aa_ref, ab_ref, s2_ref, dis_ref, b_ref, wc_ref, bc_ref, o_ref):
        dis = dis_ref[...]
        h = jnp.maximum(
            (aa_ref[...] + ab_ref[...]) * dis + s2_ref[...] + b_ref[...], 0.0)
        o_ref[...] = jnp.dot(
            h, wc_ref[...], preferred_element_type=jnp.float32) + bc_ref[...]

    g = NPAD // RB
    return pl.pallas_call(
        body,
        grid=(g,),
        in_specs=[
            pl.BlockSpec((RB, H), lambda i: (i, 0)),
            pl.BlockSpec((RB, H), lambda i: (i, 0)),
            pl.BlockSpec((RB, H), lambda i: (i, 0)),
            pl.BlockSpec((RB, 1), lambda i: (i, 0)),
            pl.BlockSpec((1, H), lambda i: (0, 0)),
            pl.BlockSpec((H, O), lambda i: (0, 0)),
            pl.BlockSpec((1, O), lambda i: (0, 0)),
        ],
        out_specs=pl.BlockSpec((RB, O), lambda i: (i, 0)),
        out_shape=jax.ShapeDtypeStruct((NPAD, O), jnp.float32),
    )(accA, accB, s2, dis, b2, Wc, bc)


def kernel(x, edge_index, W1, b1, W2, b2, Wc, bc):
    ei = edge_index.astype(jnp.int32)
    ar = jnp.arange(EPAD - E, dtype=jnp.int32)
    src3 = jnp.concatenate([ei[0], ar % NPAD]).reshape(NW, CH, CHUNK)
    dst3 = jnp.concatenate([ei[1], N + ar % (NPAD - N)]).reshape(NW, CH, CHUNK)
    x_pad = jnp.pad(x, ((0, NPAD - N), (0, 0)))

    degp = _sc_degree(dst3)
    y1, s1, dis = _tc1(x_pad, W1,
                       degp[0].reshape(NPAD, 1), degp[1].reshape(NPAD, 1))
    acc1 = _sc_scatter(y1, src3, dst3)
    y2, s2 = _tc2(acc1[0], acc1[1], s1, dis,
                  b1.reshape(1, H), W2)
    acc2 = _sc_scatter(y2, src3, dst3)
    out = _tc3(acc2[0], acc2[1], s2, dis,
               b2.reshape(1, H), Wc, bc.reshape(1, O))
    return out[:N]


# speedup vs baseline: 47.2193x; 1.0833x over previous
import functools

import jax
import jax.numpy as jnp
from jax import lax
from jax.experimental import pallas as pl
from jax.experimental.pallas import tpu as pltpu
from jax.experimental.pallas import tpu_sc as plsc

N = 10000
NPAD = 10240
E = 320000
H = 32
F = 128
O = 4

NC = 2
NS = 16
NW = NC * NS
CHUNK = 128
CH = 79
EPAD = NW * CH * CHUNK
ROWS_PER_TILE = NPAD // NS
RB = 512
NBUF = 8
DEEP = 3


def _sc_mesh():
    return plsc.VectorSubcoreMesh(core_axis_name="c", subcore_axis_name="s")


@jax.jit
def _sc_degree(dst3):
    @functools.partial(
        pl.kernel,
        mesh=_sc_mesh(),
        out_type=jax.ShapeDtypeStruct((NC, NPAD), jnp.float32),
        scratch_types=[
            pltpu.VMEM((CH, CHUNK), jnp.int32),
            pltpu.VMEM((CHUNK,), jnp.float32),
            pltpu.VMEM((CHUNK,), jnp.float32),
            pltpu.VMEM_SHARED((NPAD,), jnp.float32),
            pltpu.SemaphoreType.DMA,
        ],
        compiler_params=pltpu.CompilerParams(use_tc_tiling_on_sc=False),
    )
    def k(dst_hbm, out_hbm, dst_v, ones_v, zeros_v, deg_sh, sem_s):
        c = lax.axis_index("c")
        s = lax.axis_index("s")
        wid = c * NS + s

        def init(i, _):
            ones_v[pl.ds(i * 16, 16)] = jnp.ones((16,), jnp.float32)
            zeros_v[pl.ds(i * 16, 16)] = jnp.zeros((16,), jnp.float32)
            return 0

        lax.fori_loop(0, CHUNK // 16, init, 0)

        def zblk(i, _):
            pltpu.async_copy(
                zeros_v, deg_sh.at[pl.ds(s * ROWS_PER_TILE + i * CHUNK, CHUNK)],
                sem_s)
            return 0

        lax.fori_loop(0, ROWS_PER_TILE // CHUNK, zblk, 0)
        pltpu.sync_copy(dst_hbm.at[wid], dst_v)

        def zdrain(i, _):
            pltpu.make_async_copy(
                zeros_v, deg_sh.at[pl.ds(s * ROWS_PER_TILE, CHUNK)],
                sem_s).wait()
            return 0

        lax.fori_loop(0, ROWS_PER_TILE // CHUNK, zdrain, 0)
        plsc.subcore_barrier()

        def body(j, _):
            pltpu.async_copy(ones_v, deg_sh.at[dst_v.at[j]], sem_s, add=True)
            return 0

        lax.fori_loop(0, CH, body, 0)

        def drain(j, _):
            pltpu.make_async_copy(ones_v, deg_sh.at[dst_v.at[0]], sem_s).wait()
            return 0

        lax.fori_loop(0, CH, drain, 0)
        plsc.subcore_barrier()
        pltpu.sync_copy(deg_sh.at[pl.ds(s * ROWS_PER_TILE, ROWS_PER_TILE)],
                        out_hbm.at[c, pl.ds(s * ROWS_PER_TILE, ROWS_PER_TILE)])

    return k(dst3)


@jax.jit
def _sc_scatter(y, src3, dst3):
    @functools.partial(
        pl.kernel,
        mesh=_sc_mesh(),
        out_type=jax.ShapeDtypeStruct((NC, NPAD, H), jnp.float32),
        scratch_types=[
            pltpu.VMEM((CH, CHUNK), jnp.int32),
            pltpu.VMEM((CH, CHUNK), jnp.int32),
            pltpu.VMEM((NBUF, CHUNK, H), jnp.float32),
            pltpu.VMEM((CHUNK, H), jnp.float32),
            pltpu.VMEM_SHARED((NPAD, H), jnp.float32),
            pltpu.SemaphoreType.DMA,
            pltpu.SemaphoreType.DMA,
        ],
        compiler_params=pltpu.CompilerParams(use_tc_tiling_on_sc=False),
    )
    def k(y_hbm, src_hbm, dst_hbm, out_hbm, src_v, dst_v, rows_v, zeros_v,
          acc_sh, sem_g, sem_s):
        c = lax.axis_index("c")
        s = lax.axis_index("s")
        wid = c * NS + s

        def zrow(i, _):
            zeros_v[i, pl.ds(0, 16)] = jnp.zeros((16,), jnp.float32)
            zeros_v[i, pl.ds(16, 16)] = jnp.zeros((16,), jnp.float32)
            return 0

        lax.fori_loop(0, CHUNK, zrow, 0)

        def zblk(i, _):
            pltpu.async_copy(
                zeros_v, acc_sh.at[pl.ds(s * ROWS_PER_TILE + i * CHUNK, CHUNK)],
                sem_s)
            return 0

        lax.fori_loop(0, ROWS_PER_TILE // CHUNK, zblk, 0)
        pltpu.async_copy(src_hbm.at[wid], src_v, sem_g)
        pltpu.async_copy(dst_hbm.at[wid], dst_v, sem_g)

        def zdrain(i, _):
            pltpu.make_async_copy(
                zeros_v, acc_sh.at[pl.ds(s * ROWS_PER_TILE, CHUNK)],
                sem_s).wait()
            return 0

        lax.fori_loop(0, ROWS_PER_TILE // CHUNK, zdrain, 0)
        pltpu.make_async_copy(src_hbm.at[wid], src_v, sem_g).wait()
        pltpu.make_async_copy(dst_hbm.at[wid], dst_v, sem_g).wait()
        plsc.subcore_barrier()

        for jj in range(NBUF - DEEP):
            pltpu.async_copy(y_hbm.at[src_v.at[jj]], rows_v.at[jj], sem_g)

        def body(j, _):
            p = j % NBUF
            pltpu.make_async_copy(
                y_hbm.at[src_v.at[j]], rows_v.at[p], sem_g).wait()
            pltpu.async_copy(
                rows_v.at[p], acc_sh.at[dst_v.at[j]], sem_s, add=True)

            @pl.when(j >= DEEP)
            def _():
                pltpu.make_async_copy(
                    rows_v.at[p], acc_sh.at[dst_v.at[j]], sem_s).wait()

            @pl.when(j < CH - (NBUF - DEEP))
            def _():
                pltpu.async_copy(
                    y_hbm.at[src_v.at[j + NBUF - DEEP]],
                    rows_v.at[(j + NBUF - DEEP) % NBUF], sem_g)

            return 0

        lax.fori_loop(0, CH, body, 0)

        def sdrain(j, _):
            pltpu.make_async_copy(
                rows_v.at[0], acc_sh.at[dst_v.at[0]], sem_s).wait()
            return 0

        lax.fori_loop(0, DEEP, sdrain, 0)
        plsc.subcore_barrier()
        pltpu.sync_copy(acc_sh.at[pl.ds(s * ROWS_PER_TILE, ROWS_PER_TILE)],
                        out_hbm.at[c, pl.ds(s * ROWS_PER_TILE, ROWS_PER_TILE)])

    return k(y, src3, dst3)


def _tc1(x_pad, W1, degA, degB):
    def body(x_ref, w_ref, da_ref, db_ref, y_ref, s_ref, dis_ref):
        dis = lax.rsqrt(da_ref[...] + db_ref[...] + 1.0)
        xw = jnp.dot(x_ref[...], w_ref[...], preferred_element_type=jnp.float32)
        y = xw * dis
        y_ref[...] = y
        s_ref[...] = y * dis
        dis_ref[...] = dis

    g = NPAD // RB
    return pl.pallas_call(
        body,
        grid=(g,),
        in_specs=[
            pl.BlockSpec((RB, F), lambda i: (i, 0)),
            pl.BlockSpec((F, H), lambda i: (0, 0)),
            pl.BlockSpec((RB, 1), lambda i: (i, 0)),
            pl.BlockSpec((RB, 1), lambda i: (i, 0)),
        ],
        out_specs=[
            pl.BlockSpec((RB, H), lambda i: (i, 0)),
            pl.BlockSpec((RB, H), lambda i: (i, 0)),
            pl.BlockSpec((RB, 1), lambda i: (i, 0)),
        ],
        out_shape=[
            jax.ShapeDtypeStruct((NPAD, H), jnp.float32),
            jax.ShapeDtypeStruct((NPAD, H), jnp.float32),
            jax.ShapeDtypeStruct((NPAD, 1), jnp.float32),
        ],
    )(x_pad, W1, degA, degB)


def _tc2(accA, accB, s1, dis, b1, W2):
    def body(aa_ref, ab_ref, s1_ref, dis_ref, b_ref, w_ref, y_ref, s_ref):
        dis = dis_ref[...]
        h = jnp.maximum(
            (aa_ref[...] + ab_ref[...]) * dis + s1_ref[...] + b_ref[...], 0.0)
        xw = jnp.dot(h, w_ref[...], preferred_element_type=jnp.float32)
        y = xw * dis
        y_ref[...] = y
        s_ref[...] = y * dis

    g = NPAD // RB
    return pl.pallas_call(
        body,
        grid=(g,),
        in_specs=[
            pl.BlockSpec((RB, H), lambda i: (i, 0)),
            pl.BlockSpec((RB, H), lambda i: (i, 0)),
            pl.BlockSpec((RB, H), lambda i: (i, 0)),
            pl.BlockSpec((RB, 1), lambda i: (i, 0)),
            pl.BlockSpec((1, H), lambda i: (0, 0)),
            pl.BlockSpec((H, H), lambda i: (0, 0)),
        ],
        out_specs=[
            pl.BlockSpec((RB, H), lambda i: (i, 0)),
            pl.BlockSpec((RB, H), lambda i: (i, 0)),
        ],
        out_shape=[
            jax.ShapeDtypeStruct((NPAD, H), jnp.float32),
            jax.ShapeDtypeStruct((NPAD, H), jnp.float32),
        ],
    )(accA, accB, s1, dis, b1, W2)


def _tc3(accA, accB, s2, dis, b2, Wc, bc):
    def body(aa_ref, ab_ref, s2_ref, dis_ref, b_ref, wc_ref, bc_ref, o_ref):
        dis = dis_ref[...]
        h = jnp.maximum(
            (aa_ref[...] + ab_ref[...]) * dis + s2_ref[...] + b_ref[...], 0.0)
        o_ref[...] = jnp.dot(
            h, wc_ref[...], preferred_element_type=jnp.float32) + bc_ref[...]

    g = NPAD // RB
    return pl.pallas_call(
        body,
        grid=(g,),
        in_specs=[
            pl.BlockSpec((RB, H), lambda i: (i, 0)),
            pl.BlockSpec((RB, H), lambda i: (i, 0)),
            pl.BlockSpec((RB, H), lambda i: (i, 0)),
            pl.BlockSpec((RB, 1), lambda i: (i, 0)),
            pl.BlockSpec((1, H), lambda i: (0, 0)),
            pl.BlockSpec((H, O), lambda i: (0, 0)),
            pl.BlockSpec((1, O), lambda i: (0, 0)),
        ],
        out_specs=pl.BlockSpec((RB, O), lambda i: (i, 0)),
        out_shape=jax.ShapeDtypeStruct((NPAD, O), jnp.float32),
    )(accA, accB, s2, dis, b2, Wc, bc)


def kernel(x, edge_index, W1, b1, W2, b2, Wc, bc):
    ei = edge_index.astype(jnp.int32)
    ar = jnp.arange(EPAD - E, dtype=jnp.int32)
    src3 = jnp.concatenate([ei[0], ar % NPAD]).reshape(NW, CH, CHUNK)
    dst3 = jnp.concatenate([ei[1], N + ar % (NPAD - N)]).reshape(NW, CH, CHUNK)
    x_pad = jnp.pad(x, ((0, NPAD - N), (0, 0)))

    degp = _sc_degree(dst3)
    y1, s1, dis = _tc1(x_pad, W1,
                       degp[0].reshape(NPAD, 1), degp[1].reshape(NPAD, 1))
    acc1 = _sc_scatter(y1, src3, dst3)
    y2, s2 = _tc2(acc1[0], acc1[1], s1, dis,
                  b1.reshape(1, H), W2)
    acc2 = _sc_scatter(y2, src3, dst3)
    out = _tc3(acc2[0], acc2[1], s2, dis,
               b2.reshape(1, H), Wc, bc.reshape(1, O))
    return out[:N]


# speedup vs baseline: 52.2556x; 1.1067x over previous
import functools

import jax
import jax.numpy as jnp
from jax import lax
from jax.experimental import pallas as pl
from jax.experimental.pallas import tpu as pltpu
from jax.experimental.pallas import tpu_sc as plsc

N = 10000
NPAD = 10240
E = 320000
H = 32
F = 128
O = 4

NC = 2
NS = 16
NW = NC * NS
CHUNK = 128
CH = 80
EPAD = NW * CH * CHUNK
RPT = NPAD // NS
RB = 2048
NBUF = 8
DEEP = 3

_SC_PARAMS_NL = pltpu.CompilerParams(
    use_tc_tiling_on_sc=False, needs_layout_passes=False)


def _sc_mesh():
    return plsc.VectorSubcoreMesh(core_axis_name="c", subcore_axis_name="s")


def _zero_buffers(zr_v, ones_v, z1_v):
    def init16(i, _):
        ones_v[pl.ds(i * 16, 16)] = jnp.ones((16,), jnp.float32)
        z1_v[pl.ds(i * 16, 16)] = jnp.zeros((16,), jnp.float32)
        return 0

    lax.fori_loop(0, CHUNK // 16, init16, 0)

    def zrow(i, _):
        zr_v[i, pl.ds(0, 16)] = jnp.zeros((16,), jnp.float32)
        zr_v[i, pl.ds(16, 16)] = jnp.zeros((16,), jnp.float32)
        return 0

    lax.fori_loop(0, CHUNK, zrow, 0)


def _expand_dis(dis_v, st_v):
    def expand(r, _):
        sp = plsc.load_gather(dis_v, (jnp.full((16,), r, jnp.int32),))
        st_v[r, pl.ds(0, 16)] = sp
        st_v[r, pl.ds(16, 16)] = sp
        return 0

    lax.fori_loop(0, RPT, expand, 0)


def _rowwise_mul(dst_v, a_v, b_v):
    def mul(r, _):
        dst_v[r, pl.ds(0, 16)] = a_v[r, pl.ds(0, 16)] * b_v[r, pl.ds(0, 16)]
        dst_v[r, pl.ds(16, 16)] = (
            a_v[r, pl.ds(16, 16)] * b_v[r, pl.ds(16, 16)])
        return 0

    lax.fori_loop(0, RPT, mul, 0)


def _edge_pipeline(y_ref, src_v, dst_v, rows_v, acc_sh, sem_g, sem_s):
    def rbuf(p):
        return rows_v.at[pl.ds(p * CHUNK, CHUNK)]

    for jj in range(NBUF - DEEP):
        pltpu.async_copy(y_ref.at[src_v.at[jj]], rbuf(jj), sem_g)

    def body(j, _):
        p = j % NBUF
        pltpu.make_async_copy(y_ref.at[src_v.at[j]], rbuf(p), sem_g).wait()
        pltpu.async_copy(rbuf(p), acc_sh.at[dst_v.at[j]], sem_s, add=True)

        @pl.when(j >= DEEP)
        def _():
            pltpu.make_async_copy(
                rbuf(p), acc_sh.at[dst_v.at[j]], sem_s).wait()

        @pl.when(j < CH - (NBUF - DEEP))
        def _():
            pltpu.async_copy(
                y_ref.at[src_v.at[j + NBUF - DEEP]],
                rbuf((j + NBUF - DEEP) % NBUF), sem_g)

        return 0

    lax.fori_loop(0, CH, body, 0)

    def sdrain(j, _):
        pltpu.make_async_copy(
            rbuf(0), acc_sh.at[dst_v.at[0]], sem_s).wait()
        return 0

    lax.fori_loop(0, DEEP, sdrain, 0)


def _sc_common_tail(c, s, wid, src_hbm, dst_hbm, sa_v, sb_v, rows_v, xw_v,
                    st_v, y_ref, acc_sh, p_out, sem_g, sem_s):
    _rowwise_mul(xw_v, xw_v, st_v)
    pltpu.sync_copy(xw_v, y_ref.at[pl.ds(s * RPT, RPT)])
    _rowwise_mul(xw_v, xw_v, st_v)
    plsc.subcore_barrier()

    pltpu.async_copy(src_hbm.at[wid], sa_v, sem_g)
    pltpu.async_copy(dst_hbm.at[wid], sb_v, sem_g)
    pltpu.make_async_copy(src_hbm.at[wid], sa_v, sem_g).wait()
    pltpu.make_async_copy(dst_hbm.at[wid], sb_v, sem_g).wait()
    _edge_pipeline(y_ref, sa_v, sb_v, rows_v, acc_sh, sem_g, sem_s)
    plsc.subcore_barrier()

    arow = rows_v.at[pl.ds(0, RPT)]
    pltpu.sync_copy(acc_sh.at[pl.ds(s * RPT, RPT)], arow)

    def scale(r, _):
        a = rows_v[r, pl.ds(0, 16)] * st_v[r, pl.ds(0, 16)]
        b = rows_v[r, pl.ds(16, 16)] * st_v[r, pl.ds(16, 16)]
        rows_v[r, pl.ds(0, 16)] = a
        rows_v[r, pl.ds(16, 16)] = b
        return 0

    lax.fori_loop(0, RPT, scale, 0)

    @pl.when(c == 0)
    def _():
        def selfterm(r, _):
            rows_v[r, pl.ds(0, 16)] = (
                rows_v[r, pl.ds(0, 16)] + xw_v[r, pl.ds(0, 16)])
            rows_v[r, pl.ds(16, 16)] = (
                rows_v[r, pl.ds(16, 16)] + xw_v[r, pl.ds(16, 16)])
            return 0

        lax.fori_loop(0, RPT, selfterm, 0)

    pltpu.sync_copy(arow, p_out.at[c, pl.ds(s * RPT, RPT)])


@jax.jit
def _sc_layer1(xw1, src3, dst3):
    @functools.partial(
        pl.kernel,
        mesh=_sc_mesh(),
        out_type=[
            jax.ShapeDtypeStruct((NC, NPAD, H), jnp.float32),
            jax.ShapeDtypeStruct((NPAD,), jnp.float32),
            jax.ShapeDtypeStruct((NC, NPAD, H), jnp.float32),
        ],
        scratch_types=[
            pltpu.VMEM((CH, CHUNK), jnp.int32),
            pltpu.VMEM((CH, CHUNK), jnp.int32),
            pltpu.VMEM((NBUF * CHUNK, H), jnp.float32),
            pltpu.VMEM((CHUNK, H), jnp.float32),
            pltpu.VMEM((CHUNK,), jnp.float32),
            pltpu.VMEM((CHUNK,), jnp.float32),
            pltpu.VMEM((RPT,), jnp.float32),
            pltpu.VMEM((RPT,), jnp.float32),
            pltpu.VMEM((RPT, H), jnp.float32),
            pltpu.VMEM((RPT, H), jnp.float32),
            pltpu.VMEM_SHARED((NPAD,), jnp.float32),
            pltpu.VMEM_SHARED((NPAD, H), jnp.float32),
            pltpu.SemaphoreType.DMA,
            pltpu.SemaphoreType.DMA,
        ],
        compiler_params=_SC_PARAMS_NL,
    )
    def k(xw_hbm, src_hbm, dst_hbm, p_out, dis_out, y_out,
          sa_v, sb_v, rows_v, zr_v, ones_v, z1_v, deg_v, dis_v, xw_v, st_v,
          deg_sh, acc_sh, sem_g, sem_s):
        c = lax.axis_index("c")
        s = lax.axis_index("s")
        wid = c * NS + s
        _zero_buffers(zr_v, ones_v, z1_v)

        def zacc(i, _):
            pltpu.async_copy(
                zr_v, acc_sh.at[pl.ds(s * RPT + i * CHUNK, CHUNK)], sem_s)
            pltpu.async_copy(
                z1_v, deg_sh.at[pl.ds(s * RPT + i * CHUNK, CHUNK)], sem_s)
            return 0

        lax.fori_loop(0, RPT // CHUNK, zacc, 0)
        pltpu.async_copy(dst_hbm.at[s], sa_v, sem_g)
        pltpu.async_copy(dst_hbm.at[s + NS], sb_v, sem_g)
        pltpu.async_copy(xw_hbm.at[pl.ds(s * RPT, RPT)], xw_v, sem_g)

        def zdrain(i, _):
            pltpu.make_async_copy(
                zr_v, acc_sh.at[pl.ds(s * RPT, CHUNK)], sem_s).wait()
            pltpu.make_async_copy(
                z1_v, deg_sh.at[pl.ds(s * RPT, CHUNK)], sem_s).wait()
            return 0

        lax.fori_loop(0, RPT // CHUNK, zdrain, 0)
        pltpu.make_async_copy(dst_hbm.at[s], sa_v, sem_g).wait()
        pltpu.make_async_copy(dst_hbm.at[s + NS], sb_v, sem_g).wait()
        plsc.subcore_barrier()

        def dfire(j, _):
            pltpu.async_copy(ones_v, deg_sh.at[sa_v.at[j]], sem_s, add=True)
            pltpu.async_copy(ones_v, deg_sh.at[sb_v.at[j]], sem_s, add=True)
            return 0

        lax.fori_loop(0, CH, dfire, 0)

        def ddrain(j, _):
            pltpu.make_async_copy(ones_v, deg_sh.at[sa_v.at[0]], sem_s).wait()
            pltpu.make_async_copy(ones_v, deg_sh.at[sa_v.at[0]], sem_s).wait()
            return 0

        lax.fori_loop(0, CH, ddrain, 0)
        plsc.subcore_barrier()

        pltpu.sync_copy(deg_sh.at[pl.ds(s * RPT, RPT)], deg_v)

        def newton(i, _):
            u = deg_v[pl.ds(i * 16, 16)] + 1.0
            bi = plsc.bitcast(u, jnp.int32)
            bi = jnp.int32(0x5F3759DF) - lax.shift_right_logical(bi, 1)
            yv = plsc.bitcast(bi, jnp.float32)
            yv = yv * (1.5 - 0.5 * u * yv * yv)
            yv = yv * (1.5 - 0.5 * u * yv * yv)
            yv = yv * (1.5 - 0.5 * u * yv * yv)
            dis_v[pl.ds(i * 16, 16)] = yv
            return 0

        lax.fori_loop(0, RPT // 16, newton, 0)

        @pl.when(c == 0)
        def _():
            pltpu.async_copy(dis_v, dis_out.at[pl.ds(s * RPT, RPT)], sem_s)

        _expand_dis(dis_v, st_v)
        pltpu.make_async_copy(
            xw_hbm.at[pl.ds(s * RPT, RPT)], xw_v, sem_g).wait()

        @pl.when(c == 0)
        def _():
            pltpu.make_async_copy(
                dis_v, dis_out.at[pl.ds(s * RPT, RPT)], sem_s).wait()

        _sc_common_tail(c, s, wid, src_hbm, dst_hbm, sa_v, sb_v, rows_v,
                        xw_v, st_v, y_out.at[c], acc_sh, p_out, sem_g, sem_s)

    return k(xw1, src3, dst3)


@jax.jit
def _sc_layer2(xw2, dis, src3, dst3):
    @functools.partial(
        pl.kernel,
        mesh=_sc_mesh(),
        out_type=[
            jax.ShapeDtypeStruct((NC, NPAD, H), jnp.float32),
            jax.ShapeDtypeStruct((NC, NPAD, H), jnp.float32),
        ],
        scratch_types=[
            pltpu.VMEM((CH, CHUNK), jnp.int32),
            pltpu.VMEM((CH, CHUNK), jnp.int32),
            pltpu.VMEM((NBUF * CHUNK, H), jnp.float32),
            pltpu.VMEM((CHUNK, H), jnp.float32),
            pltpu.VMEM((CHUNK,), jnp.float32),
            pltpu.VMEM((CHUNK,), jnp.float32),
            pltpu.VMEM((RPT,), jnp.float32),
            pltpu.VMEM((RPT, H), jnp.float32),
            pltpu.VMEM((RPT, H), jnp.float32),
            pltpu.VMEM_SHARED((NPAD, H), jnp.float32),
            pltpu.SemaphoreType.DMA,
            pltpu.SemaphoreType.DMA,
        ],
        compiler_params=_SC_PARAMS_NL,
    )
    def k(xw_hbm, dis_hbm, src_hbm, dst_hbm, p_out, y_out,
          sa_v, sb_v, rows_v, zr_v, ones_v, z1_v, dis_v, xw_v, st_v,
          acc_sh, sem_g, sem_s):
        c = lax.axis_index("c")
        s = lax.axis_index("s")
        wid = c * NS + s
        _zero_buffers(zr_v, ones_v, z1_v)

        def zacc(i, _):
            pltpu.async_copy(
                zr_v, acc_sh.at[pl.ds(s * RPT + i * CHUNK, CHUNK)], sem_s)
            return 0

        lax.fori_loop(0, RPT // CHUNK, zacc, 0)
        pltpu.async_copy(dis_hbm.at[pl.ds(s * RPT, RPT)], dis_v, sem_g)
        pltpu.async_copy(xw_hbm.at[pl.ds(s * RPT, RPT)], xw_v, sem_g)

        def zdrain(i, _):
            pltpu.make_async_copy(
                zr_v, acc_sh.at[pl.ds(s * RPT, CHUNK)], sem_s).wait()
            return 0

        lax.fori_loop(0, RPT // CHUNK, zdrain, 0)
        pltpu.make_async_copy(
            dis_hbm.at[pl.ds(s * RPT, RPT)], dis_v, sem_g).wait()
        _expand_dis(dis_v, st_v)
        pltpu.make_async_copy(
            xw_hbm.at[pl.ds(s * RPT, RPT)], xw_v, sem_g).wait()
        _sc_common_tail(c, s, wid, src_hbm, dst_hbm, sa_v, sb_v, rows_v,
                        xw_v, st_v, y_out.at[c], acc_sh, p_out, sem_g, sem_s)

    return k(xw2, dis, src3, dst3)


def _tc_xw1(x_pad, W1):
    def body(x_ref, w_ref, o_ref):
        o_ref[...] = jnp.dot(x_ref[...], w_ref[...],
                             preferred_element_type=jnp.float32)

    return pl.pallas_call(
        body,
        grid=(NPAD // RB,),
        in_specs=[
            pl.BlockSpec((RB, F), lambda i: (i, 0)),
            pl.BlockSpec((F, H), lambda i: (0, 0)),
        ],
        out_specs=pl.BlockSpec((RB, H), lambda i: (i, 0)),
        out_shape=jax.ShapeDtypeStruct((NPAD, H), jnp.float32),
    )(x_pad, W1)


def _tc_mid(p1, b1, W2):
    def body(a0_ref, a1_ref, b_ref, w_ref, o_ref):
        h = jnp.maximum(a0_ref[0] + a1_ref[0] + b_ref[...], 0.0)
        o_ref[...] = jnp.dot(h, w_ref[...],
                             preferred_element_type=jnp.float32)

    return pl.pallas_call(
        body,
        grid=(NPAD // RB,),
        in_specs=[
            pl.BlockSpec((1, RB, H), lambda i: (0, i, 0)),
            pl.BlockSpec((1, RB, H), lambda i: (1, i, 0)),
            pl.BlockSpec((1, H), lambda i: (0, 0)),
            pl.BlockSpec((H, H), lambda i: (0, 0)),
        ],
        out_specs=pl.BlockSpec((RB, H), lambda i: (i, 0)),
        out_shape=jax.ShapeDtypeStruct((NPAD, H), jnp.float32),
    )(p1, p1, b1, W2)


def _tc_out(p2, b2, Wc, bc):
    def body(a0_ref, a1_ref, b_ref, wc_ref, bc_ref, o_ref):
        h = jnp.maximum(a0_ref[0] + a1_ref[0] + b_ref[...], 0.0)
        o_ref[...] = jnp.dot(
            h, wc_ref[...], preferred_element_type=jnp.float32) + bc_ref[...]

    return pl.pallas_call(
        body,
        grid=(NPAD // RB,),
        in_specs=[
            pl.BlockSpec((1, RB, H), lambda i: (0, i, 0)),
            pl.BlockSpec((1, RB, H), lambda i: (1, i, 0)),
            pl.BlockSpec((1, H), lambda i: (0, 0)),
            pl.BlockSpec((H, O), lambda i: (0, 0)),
            pl.BlockSpec((1, O), lambda i: (0, 0)),
        ],
        out_specs=pl.BlockSpec((RB, O), lambda i: (i, 0)),
        out_shape=jax.ShapeDtypeStruct((NPAD, O), jnp.float32),
    )(p2, p2, b2, Wc, bc)


def kernel(x, edge_index, W1, b1, W2, b2, Wc, bc):
    ei = edge_index.astype(jnp.int32)
    ar = jnp.arange(EPAD - E, dtype=jnp.int32)
    src3 = jnp.concatenate([ei[0], ar % NPAD]).reshape(NW, CH, CHUNK)
    dst3 = jnp.concatenate([ei[1], N + ar % (NPAD - N)]).reshape(NW, CH, CHUNK)
    x_pad = jnp.pad(x, ((0, NPAD - N), (0, 0)))

    xw1 = _tc_xw1(x_pad, W1)
    p1, dis, _y1 = _sc_layer1(xw1, src3, dst3)
    xw2 = _tc_mid(p1, b1.reshape(1, H), W2)
    p2, _y2 = _sc_layer2(xw2, dis, src3, dst3)
    out = _tc_out(p2, b2.reshape(1, H), Wc, bc.reshape(1, O))
    return out[:N]


# speedup vs baseline: 56.8913x; 1.0887x over previous
import functools

import jax
import jax.numpy as jnp
from jax import lax
from jax.experimental import pallas as pl
from jax.experimental.pallas import tpu as pltpu
from jax.experimental.pallas import tpu_sc as plsc

N = 10000
NPAD = 10240
E = 320000
H = 32
F = 128
O = 4

NC = 2
NS = 16
NW = NC * NS
CHUNK = 128
CH = 80
SLAB = CH * CHUNK
CH_LAST = (E - (NW - 1) * SLAB) // CHUNK
RPT = NPAD // NS
RB = 2048
NBUF = 8
DEEP = 3

_SC_PARAMS_NL = pltpu.CompilerParams(
    use_tc_tiling_on_sc=False, needs_layout_passes=False)


def _sc_mesh():
    return plsc.VectorSubcoreMesh(core_axis_name="c", subcore_axis_name="s")


def _zero_buffers(zr_v, ones_v, z1_v):
    def init16(i, _):
        ones_v[pl.ds(i * 16, 16)] = jnp.ones((16,), jnp.float32)
        z1_v[pl.ds(i * 16, 16)] = jnp.zeros((16,), jnp.float32)
        return 0

    lax.fori_loop(0, CHUNK // 16, init16, 0)

    def zrow(i, _):
        zr_v[i, pl.ds(0, 16)] = jnp.zeros((16,), jnp.float32)
        zr_v[i, pl.ds(16, 16)] = jnp.zeros((16,), jnp.float32)
        return 0

    lax.fori_loop(0, CHUNK, zrow, 0)


def _expand_dis(dis_v, st_v):
    def expand(r, _):
        sp = plsc.load_gather(dis_v, (jnp.full((16,), r, jnp.int32),))
        st_v[r, pl.ds(0, 16)] = sp
        st_v[r, pl.ds(16, 16)] = sp
        return 0

    lax.fori_loop(0, RPT, expand, 0)


def _rowwise_mul(dst_v, a_v, b_v):
    def mul(r, _):
        dst_v[r, pl.ds(0, 16)] = a_v[r, pl.ds(0, 16)] * b_v[r, pl.ds(0, 16)]
        dst_v[r, pl.ds(16, 16)] = (
            a_v[r, pl.ds(16, 16)] * b_v[r, pl.ds(16, 16)])
        return 0

    lax.fori_loop(0, RPT, mul, 0)


def _edge_pipeline(y_ref, src_v, dst_v, rows_v, acc_sh, sem_g, sem_s, nch):
    def rbuf(p):
        return rows_v.at[pl.ds(p * CHUNK, CHUNK)]

    for jj in range(NBUF - DEEP):
        pltpu.async_copy(y_ref.at[src_v.at[jj]], rbuf(jj), sem_g)

    def body(j, _):
        p = j % NBUF
        pltpu.make_async_copy(y_ref.at[src_v.at[j]], rbuf(p), sem_g).wait()
        pltpu.async_copy(rbuf(p), acc_sh.at[dst_v.at[j]], sem_s, add=True)

        @pl.when(j >= DEEP)
        def _():
            pltpu.make_async_copy(
                rbuf(p), acc_sh.at[dst_v.at[j]], sem_s).wait()

        @pl.when(j < nch - (NBUF - DEEP))
        def _():
            pltpu.async_copy(
                y_ref.at[src_v.at[j + NBUF - DEEP]],
                rbuf((j + NBUF - DEEP) % NBUF), sem_g)

        return 0

    lax.fori_loop(0, nch, body, 0)

    def sdrain(j, _):
        pltpu.make_async_copy(
            rbuf(0), acc_sh.at[dst_v.at[0]], sem_s).wait()
        return 0

    lax.fori_loop(0, DEEP, sdrain, 0)


def _load_slab(ei_hbm, row, base, n, idx_v, sem):
    def ld(j, _):
        pltpu.async_copy(
            ei_hbm.at[row, pl.ds(base + j * CHUNK, CHUNK)], idx_v.at[j], sem)
        return 0

    lax.fori_loop(0, n, ld, 0)


def _drain_slab(ei_hbm, idx_v, n, sem):
    def dr(j, _):
        pltpu.make_async_copy(
            ei_hbm.at[0, pl.ds(0, CHUNK)], idx_v.at[0], sem).wait()
        return 0

    lax.fori_loop(0, n, dr, 0)


def _sc_common_tail(c, s, wid, ei_hbm, sa_v, sb_v, rows_v, xw_v,
                    st_v, y_ref, acc_sh, p_out, sem_g, sem_s):
    _rowwise_mul(xw_v, xw_v, st_v)
    pltpu.sync_copy(xw_v, y_ref.at[pl.ds(s * RPT, RPT)])
    _rowwise_mul(xw_v, xw_v, st_v)
    plsc.subcore_barrier()

    base = wid * SLAB
    nch = jnp.where(wid == NW - 1, CH_LAST, CH)
    _load_slab(ei_hbm, 0, base, nch, sa_v, sem_g)
    _load_slab(ei_hbm, 1, base, nch, sb_v, sem_g)
    _drain_slab(ei_hbm, sa_v, 2 * nch, sem_g)
    _edge_pipeline(y_ref, sa_v, sb_v, rows_v, acc_sh, sem_g, sem_s, nch)
    plsc.subcore_barrier()

    arow = rows_v.at[pl.ds(0, RPT)]
    pltpu.sync_copy(acc_sh.at[pl.ds(s * RPT, RPT)], arow)

    def scale(r, _):
        a = rows_v[r, pl.ds(0, 16)] * st_v[r, pl.ds(0, 16)]
        b = rows_v[r, pl.ds(16, 16)] * st_v[r, pl.ds(16, 16)]
        rows_v[r, pl.ds(0, 16)] = a
        rows_v[r, pl.ds(16, 16)] = b
        return 0

    lax.fori_loop(0, RPT, scale, 0)

    @pl.when(c == 0)
    def _():
        def selfterm(r, _):
            rows_v[r, pl.ds(0, 16)] = (
                rows_v[r, pl.ds(0, 16)] + xw_v[r, pl.ds(0, 16)])
            rows_v[r, pl.ds(16, 16)] = (
                rows_v[r, pl.ds(16, 16)] + xw_v[r, pl.ds(16, 16)])
            return 0

        lax.fori_loop(0, RPT, selfterm, 0)

    pltpu.sync_copy(arow, p_out.at[c, pl.ds(s * RPT, RPT)])


@jax.jit
def _sc_layer1(xw1, ei):
    @functools.partial(
        pl.kernel,
        mesh=_sc_mesh(),
        out_type=[
            jax.ShapeDtypeStruct((NC, NPAD, H), jnp.float32),
            jax.ShapeDtypeStruct((NPAD,), jnp.float32),
            jax.ShapeDtypeStruct((NC, NPAD, H), jnp.float32),
        ],
        scratch_types=[
            pltpu.VMEM((CH, CHUNK), jnp.int32),
            pltpu.VMEM((CH, CHUNK), jnp.int32),
            pltpu.VMEM((NBUF * CHUNK, H), jnp.float32),
            pltpu.VMEM((CHUNK, H), jnp.float32),
            pltpu.VMEM((CHUNK,), jnp.float32),
            pltpu.VMEM((CHUNK,), jnp.float32),
            pltpu.VMEM((RPT,), jnp.float32),
            pltpu.VMEM((RPT,), jnp.float32),
            pltpu.VMEM((RPT, H), jnp.float32),
            pltpu.VMEM((RPT, H), jnp.float32),
            pltpu.VMEM_SHARED((NPAD,), jnp.float32),
            pltpu.VMEM_SHARED((NPAD, H), jnp.float32),
            pltpu.SemaphoreType.DMA,
            pltpu.SemaphoreType.DMA,
            pltpu.SemaphoreType.DMA,
        ],
        compiler_params=_SC_PARAMS_NL,
    )
    def k(xw_hbm, ei_hbm, p_out, dis_out, y_out,
          sa_v, sb_v, rows_v, zr_v, ones_v, z1_v, deg_v, dis_v, xw_v, st_v,
          deg_sh, acc_sh, sem_g, sem_s, sem_x):
        c = lax.axis_index("c")
        s = lax.axis_index("s")
        wid = c * NS + s
        _zero_buffers(zr_v, ones_v, z1_v)

        def zacc(i, _):
            pltpu.async_copy(
                zr_v, acc_sh.at[pl.ds(s * RPT + i * CHUNK, CHUNK)], sem_s)
            pltpu.async_copy(
                z1_v, deg_sh.at[pl.ds(s * RPT + i * CHUNK, CHUNK)], sem_s)
            return 0

        lax.fori_loop(0, RPT // CHUNK, zacc, 0)
        nb = jnp.where(s == NS - 1, CH_LAST, CH)
        _load_slab(ei_hbm, 1, s * SLAB, CH, sa_v, sem_g)
        _load_slab(ei_hbm, 1, (s + NS) * SLAB, nb, sb_v, sem_g)
        pltpu.async_copy(xw_hbm.at[pl.ds(s * RPT, RPT)], xw_v, sem_x)

        def zdrain(i, _):
            pltpu.make_async_copy(
                zr_v, acc_sh.at[pl.ds(s * RPT, CHUNK)], sem_s).wait()
            pltpu.make_async_copy(
                z1_v, deg_sh.at[pl.ds(s * RPT, CHUNK)], sem_s).wait()
            return 0

        lax.fori_loop(0, RPT // CHUNK, zdrain, 0)
        _drain_slab(ei_hbm, sa_v, CH + nb, sem_g)
        plsc.subcore_barrier()

        def dfireA(j, _):
            pltpu.async_copy(ones_v, deg_sh.at[sa_v.at[j]], sem_s, add=True)
            return 0

        def dfireB(j, _):
            pltpu.async_copy(ones_v, deg_sh.at[sb_v.at[j]], sem_s, add=True)
            return 0

        lax.fori_loop(0, CH, dfireA, 0)
        lax.fori_loop(0, nb, dfireB, 0)

        def ddrain(j, _):
            pltpu.make_async_copy(ones_v, deg_sh.at[sa_v.at[0]], sem_s).wait()
            return 0

        lax.fori_loop(0, CH + nb, ddrain, 0)
        plsc.subcore_barrier()

        pltpu.sync_copy(deg_sh.at[pl.ds(s * RPT, RPT)], deg_v)

        def newton(i, _):
            u = deg_v[pl.ds(i * 16, 16)] + 1.0
            bi = plsc.bitcast(u, jnp.int32)
            bi = jnp.int32(0x5F3759DF) - lax.shift_right_logical(bi, 1)
            yv = plsc.bitcast(bi, jnp.float32)
            yv = yv * (1.5 - 0.5 * u * yv * yv)
            yv = yv * (1.5 - 0.5 * u * yv * yv)
            yv = yv * (1.5 - 0.5 * u * yv * yv)
            dis_v[pl.ds(i * 16, 16)] = yv
            return 0

        lax.fori_loop(0, RPT // 16, newton, 0)

        @pl.when(c == 0)
        def _():
            pltpu.async_copy(dis_v, dis_out.at[pl.ds(s * RPT, RPT)], sem_s)

        _expand_dis(dis_v, st_v)
        pltpu.make_async_copy(
            xw_hbm.at[pl.ds(s * RPT, RPT)], xw_v, sem_x).wait()

        @pl.when(c == 0)
        def _():
            pltpu.make_async_copy(
                dis_v, dis_out.at[pl.ds(s * RPT, RPT)], sem_s).wait()

        _sc_common_tail(c, s, wid, ei_hbm, sa_v, sb_v, rows_v,
                        xw_v, st_v, y_out.at[c], acc_sh, p_out, sem_g, sem_s)

    return k(xw1, ei)


@jax.jit
def _sc_layer2(xw2, dis, ei):
    @functools.partial(
        pl.kernel,
        mesh=_sc_mesh(),
        out_type=[
            jax.ShapeDtypeStruct((NC, NPAD, H), jnp.float32),
            jax.ShapeDtypeStruct((NC, NPAD, H), jnp.float32),
        ],
        scratch_types=[
            pltpu.VMEM((CH, CHUNK), jnp.int32),
            pltpu.VMEM((CH, CHUNK), jnp.int32),
            pltpu.VMEM((NBUF * CHUNK, H), jnp.float32),
            pltpu.VMEM((CHUNK, H), jnp.float32),
            pltpu.VMEM((CHUNK,), jnp.float32),
            pltpu.VMEM((CHUNK,), jnp.float32),
            pltpu.VMEM((RPT,), jnp.float32),
            pltpu.VMEM((RPT, H), jnp.float32),
            pltpu.VMEM((RPT, H), jnp.float32),
            pltpu.VMEM_SHARED((NPAD, H), jnp.float32),
            pltpu.SemaphoreType.DMA,
            pltpu.SemaphoreType.DMA,
            pltpu.SemaphoreType.DMA,
        ],
        compiler_params=_SC_PARAMS_NL,
    )
    def k(xw_hbm, dis_hbm, ei_hbm, p_out, y_out,
          sa_v, sb_v, rows_v, zr_v, ones_v, z1_v, dis_v, xw_v, st_v,
          acc_sh, sem_g, sem_s, sem_x):
        c = lax.axis_index("c")
        s = lax.axis_index("s")
        wid = c * NS + s
        _zero_buffers(zr_v, ones_v, z1_v)

        def zacc(i, _):
            pltpu.async_copy(
                zr_v, acc_sh.at[pl.ds(s * RPT + i * CHUNK, CHUNK)], sem_s)
            return 0

        lax.fori_loop(0, RPT // CHUNK, zacc, 0)
        pltpu.async_copy(dis_hbm.at[pl.ds(s * RPT, RPT)], dis_v, sem_x)
        pltpu.async_copy(xw_hbm.at[pl.ds(s * RPT, RPT)], xw_v, sem_x)

        def zdrain(i, _):
            pltpu.make_async_copy(
                zr_v, acc_sh.at[pl.ds(s * RPT, CHUNK)], sem_s).wait()
            return 0

        lax.fori_loop(0, RPT // CHUNK, zdrain, 0)
        pltpu.make_async_copy(
            dis_hbm.at[pl.ds(s * RPT, RPT)], dis_v, sem_x).wait()
        pltpu.make_async_copy(
            xw_hbm.at[pl.ds(s * RPT, RPT)], xw_v, sem_x).wait()
        _expand_dis(dis_v, st_v)
        _sc_common_tail(c, s, wid, ei_hbm, sa_v, sb_v, rows_v,
                        xw_v, st_v, y_out.at[c], acc_sh, p_out, sem_g, sem_s)

    return k(xw2, dis, ei)


def _tc_xw1(x, W1):
    def body(x_ref, w_ref, o_ref):
        o_ref[...] = jnp.dot(x_ref[...], w_ref[...],
                             preferred_element_type=jnp.float32)

    return pl.pallas_call(
        body,
        grid=(NPAD // RB,),
        in_specs=[
            pl.BlockSpec((RB, F), lambda i: (i, 0)),
            pl.BlockSpec((F, H), lambda i: (0, 0)),
        ],
        out_specs=pl.BlockSpec((RB, H), lambda i: (i, 0)),
        out_shape=jax.ShapeDtypeStruct((NPAD, H), jnp.float32),
    )(x, W1)


def _tc_mid(p1, b1, W2):
    def body(a0_ref, a1_ref, b_ref, w_ref, o_ref):
        h = jnp.maximum(a0_ref[0] + a1_ref[0] + b_ref[...], 0.0)
        o_ref[...] = jnp.dot(h, w_ref[...],
                             preferred_element_type=jnp.float32)

    return pl.pallas_call(
        body,
        grid=(NPAD // RB,),
        in_specs=[
            pl.BlockSpec((1, RB, H), lambda i: (0, i, 0)),
            pl.BlockSpec((1, RB, H), lambda i: (1, i, 0)),
            pl.BlockSpec((1, H), lambda i: (0, 0)),
            pl.BlockSpec((H, H), lambda i: (0, 0)),
        ],
        out_specs=pl.BlockSpec((RB, H), lambda i: (i, 0)),
        out_shape=jax.ShapeDtypeStruct((NPAD, H), jnp.float32),
    )(p1, p1, b1, W2)


def _tc_out(p2, b2, Wc, bc):
    def body(a0_ref, a1_ref, b_ref, wc_ref, bc_ref, o_ref):
        h = jnp.maximum(a0_ref[0] + a1_ref[0] + b_ref[...], 0.0)
        o_ref[...] = jnp.dot(
            h, wc_ref[...], preferred_element_type=jnp.float32) + bc_ref[...]

    return pl.pallas_call(
        body,
        grid=(NPAD // RB,),
        in_specs=[
            pl.BlockSpec((1, RB, H), lambda i: (0, i, 0)),
            pl.BlockSpec((1, RB, H), lambda i: (1, i, 0)),
            pl.BlockSpec((1, H), lambda i: (0, 0)),
            pl.BlockSpec((H, O), lambda i: (0, 0)),
            pl.BlockSpec((1, O), lambda i: (0, 0)),
        ],
        out_specs=pl.BlockSpec((RB, O), lambda i: (i, 0)),
        out_shape=jax.ShapeDtypeStruct((N, O), jnp.float32),
    )(p2, p2, b2, Wc, bc)


def kernel(x, edge_index, W1, b1, W2, b2, Wc, bc):
    ei = edge_index.astype(jnp.int32)
    xw1 = _tc_xw1(x, W1)
    p1, dis, _y1 = _sc_layer1(xw1, ei)
    xw2 = _tc_mid(p1, b1.reshape(1, H), W2)
    p2, _y2 = _sc_layer2(xw2, dis, ei)
    return _tc_out(p2, b2.reshape(1, H), Wc, bc.reshape(1, O))


# speedup vs baseline: 60.3922x; 1.0615x over previous
import functools

import jax
import jax.numpy as jnp
from jax import lax
from jax.experimental import pallas as pl
from jax.experimental.pallas import tpu as pltpu
from jax.experimental.pallas import tpu_sc as plsc

N = 10000
NPAD = 10240
E = 320000
H = 32
F = 128
O = 4

NC = 2
NS = 16
NW = NC * NS
CHUNK = 128
CH = 80
SLAB = CH * CHUNK
CH_LAST = (E - (NW - 1) * SLAB) // CHUNK
RPT = NPAD // NS
RB = 2048
PH = NPAD // 4
PRT = RPT // 4
RBQ = RB // 4
O4 = 4 * O
NG = RPT * H // 16
NBUF = 8
DEEP = 3

_SC_PARAMS_NL = pltpu.CompilerParams(
    use_tc_tiling_on_sc=False, needs_layout_passes=False)


def _sc_mesh():
    return plsc.VectorSubcoreMesh(core_axis_name="c", subcore_axis_name="s")


def _zero_buffers(zr_v, ones_v, z1_v):
    def init16(i, _):
        ones_v[pl.ds(i * 16, 16)] = jnp.ones((16,), jnp.float32)
        z1_v[pl.ds(i * 16, 16)] = jnp.zeros((16,), jnp.float32)
        return 0

    lax.fori_loop(0, CHUNK // 16, init16, 0)

    def zrow(i, _):
        zr_v[i, pl.ds(0, 16)] = jnp.zeros((16,), jnp.float32)
        zr_v[i, pl.ds(16, 16)] = jnp.zeros((16,), jnp.float32)
        return 0

    lax.fori_loop(0, CHUNK, zrow, 0)


def _splat(dis_v, node):
    return plsc.load_gather(dis_v, (jnp.full((16,), node, jnp.int32),))


def _edge_pipeline(y_ref, src_v, dst_v, rows_v, acc_sh, sem_g, sem_s, nch):
    def rbuf(p):
        return rows_v.at[pl.ds(p * CHUNK, CHUNK)]

    for jj in range(NBUF - DEEP):
        pltpu.async_copy(y_ref.at[src_v.at[jj]], rbuf(jj), sem_g)

    def body(j, _):
        p = j % NBUF
        pltpu.make_async_copy(y_ref.at[src_v.at[j]], rbuf(p), sem_g).wait()
        pltpu.async_copy(rbuf(p), acc_sh.at[dst_v.at[j]], sem_s, add=True)

        @pl.when(j >= DEEP)
        def _():
            pltpu.make_async_copy(
                rbuf(p), acc_sh.at[dst_v.at[j]], sem_s).wait()

        @pl.when(j < nch - (NBUF - DEEP))
        def _():
            pltpu.async_copy(
                y_ref.at[src_v.at[j + NBUF - DEEP]],
                rbuf((j + NBUF - DEEP) % NBUF), sem_g)

        return 0

    lax.fori_loop(0, nch, body, 0)

    def sdrain(j, _):
        pltpu.make_async_copy(
            rbuf(0), acc_sh.at[dst_v.at[0]], sem_s).wait()
        return 0

    lax.fori_loop(0, DEEP, sdrain, 0)


def _load_slab(ei_hbm, row, base, n, idx_v, sem):
    def ld(j, _):
        pltpu.async_copy(
            ei_hbm.at[row, pl.ds(base + j * CHUNK, CHUNK)], idx_v.at[j], sem)
        return 0

    lax.fori_loop(0, n, ld, 0)


def _drain_slab(ei_hbm, idx_v, n, sem):
    def dr(j, _):
        pltpu.make_async_copy(
            ei_hbm.at[0, pl.ds(0, CHUNK)], idx_v.at[0], sem).wait()
        return 0

    lax.fori_loop(0, n, dr, 0)


def _sc_common_tail(c, s, wid, ei_hbm, sa_v, sb_v, rows_v, xw_v, gw,
                    st_v, dis_v, y_ref, acc_sh, p_out, sem_g, sem_s):
    def yscale(g, _):
        sp = _splat(dis_v, g // 2)
        yv = xw_v[g // gw, pl.ds((g % gw) * 16, 16)] * sp
        rows_v[g // 2, pl.ds((g % 2) * 16, 16)] = yv
        return 0

    lax.fori_loop(0, NG, yscale, 0)
    pltpu.sync_copy(rows_v.at[pl.ds(0, RPT)], y_ref.at[pl.ds(s * RPT, RPT)])

    @pl.when(c == 0)
    def _():
        def selfterm(g, _):
            sp = _splat(dis_v, g // 2)
            st_v[g // 8, pl.ds((g % 8) * 16, 16)] = (
                rows_v[g // 2, pl.ds((g % 2) * 16, 16)] * sp)
            return 0

        lax.fori_loop(0, NG, selfterm, 0)

    plsc.subcore_barrier()

    base = wid * SLAB
    nch = jnp.where(wid == NW - 1, CH_LAST, CH)
    _load_slab(ei_hbm, 0, base, nch, sa_v, sem_g)
    _load_slab(ei_hbm, 1, base, nch, sb_v, sem_g)
    _drain_slab(ei_hbm, sa_v, 2 * nch, sem_g)
    _edge_pipeline(y_ref, sa_v, sb_v, rows_v, acc_sh, sem_g, sem_s, nch)
    plsc.subcore_barrier()

    pltpu.sync_copy(acc_sh.at[pl.ds(s * RPT, RPT)], rows_v.at[pl.ds(0, RPT)])

    def scale0(g, _):
        sp = _splat(dis_v, g // 2)
        v = (rows_v[g // 2, pl.ds((g % 2) * 16, 16)] * sp
             + st_v[g // 8, pl.ds((g % 8) * 16, 16)])
        st_v[g // 8, pl.ds((g % 8) * 16, 16)] = v
        return 0

    def scale1(g, _):
        sp = _splat(dis_v, g // 2)
        st_v[g // 8, pl.ds((g % 8) * 16, 16)] = (
            rows_v[g // 2, pl.ds((g % 2) * 16, 16)] * sp)
        return 0

    @pl.when(c == 0)
    def _():
        lax.fori_loop(0, NG, scale0, 0)

    @pl.when(c != 0)
    def _():
        lax.fori_loop(0, NG, scale1, 0)

    pltpu.sync_copy(st_v, p_out.at[c, pl.ds(s * PRT, PRT)])


@jax.jit
def _sc_layer1(xw1, ei):
    @functools.partial(
        pl.kernel,
        mesh=_sc_mesh(),
        out_type=[
            jax.ShapeDtypeStruct((NC, PH, 4 * H), jnp.float32),
            jax.ShapeDtypeStruct((NPAD,), jnp.float32),
            jax.ShapeDtypeStruct((NC, NPAD, H), jnp.float32),
        ],
        scratch_types=[
            pltpu.VMEM((CH, CHUNK), jnp.int32),
            pltpu.VMEM((CH, CHUNK), jnp.int32),
            pltpu.VMEM((NBUF * CHUNK, H), jnp.float32),
            pltpu.VMEM((CHUNK, H), jnp.float32),
            pltpu.VMEM((CHUNK,), jnp.float32),
            pltpu.VMEM((CHUNK,), jnp.float32),
            pltpu.VMEM((RPT,), jnp.float32),
            pltpu.VMEM((RPT,), jnp.float32),
            pltpu.VMEM((RPT, H), jnp.float32),
            pltpu.VMEM((PRT, 4 * H), jnp.float32),
            pltpu.VMEM_SHARED((NPAD,), jnp.float32),
            pltpu.VMEM_SHARED((NPAD, H), jnp.float32),
            pltpu.SemaphoreType.DMA,
            pltpu.SemaphoreType.DMA,
            pltpu.SemaphoreType.DMA,
        ],
        compiler_params=_SC_PARAMS_NL,
    )
    def k(xw_hbm, ei_hbm, p_out, dis_out, y_out,
          sa_v, sb_v, rows_v, zr_v, ones_v, z1_v, deg_v, dis_v, xw_v, st_v,
          deg_sh, acc_sh, sem_g, sem_s, sem_x):
        c = lax.axis_index("c")
        s = lax.axis_index("s")
        wid = c * NS + s
        _zero_buffers(zr_v, ones_v, z1_v)

        def zacc(i, _):
            pltpu.async_copy(
                zr_v, acc_sh.at[pl.ds(s * RPT + i * CHUNK, CHUNK)], sem_s)
            pltpu.async_copy(
                z1_v, deg_sh.at[pl.ds(s * RPT + i * CHUNK, CHUNK)], sem_s)
            return 0

        lax.fori_loop(0, RPT // CHUNK, zacc, 0)
        nb = jnp.where(s == NS - 1, CH_LAST, CH)
        _load_slab(ei_hbm, 1, s * SLAB, CH, sa_v, sem_g)
        _load_slab(ei_hbm, 1, (s + NS) * SLAB, nb, sb_v, sem_g)
        pltpu.async_copy(xw_hbm.at[pl.ds(s * RPT, RPT)], xw_v, sem_x)

        def zdrain(i, _):
            pltpu.make_async_copy(
                zr_v, acc_sh.at[pl.ds(s * RPT, CHUNK)], sem_s).wait()
            pltpu.make_async_copy(
                z1_v, deg_sh.at[pl.ds(s * RPT, CHUNK)], sem_s).wait()
            return 0

        lax.fori_loop(0, RPT // CHUNK, zdrain, 0)
        _drain_slab(ei_hbm, sa_v, CH + nb, sem_g)
        plsc.subcore_barrier()

        def dfireA(j, _):
            pltpu.async_copy(ones_v, deg_sh.at[sa_v.at[j]], sem_s, add=True)
            return 0

        def dfireB(j, _):
            pltpu.async_copy(ones_v, deg_sh.at[sb_v.at[j]], sem_s, add=True)
            return 0

        lax.fori_loop(0, CH, dfireA, 0)
        lax.fori_loop(0, nb, dfireB, 0)

        def ddrain(j, _):
            pltpu.make_async_copy(ones_v, deg_sh.at[sa_v.at[0]], sem_s).wait()
            return 0

        lax.fori_loop(0, CH + nb, ddrain, 0)
        plsc.subcore_barrier()

        pltpu.sync_copy(deg_sh.at[pl.ds(s * RPT, RPT)], deg_v)

        def newton(i, _):
            u = deg_v[pl.ds(i * 16, 16)] + 1.0
            bi = plsc.bitcast(u, jnp.int32)
            bi = jnp.int32(0x5F3759DF) - lax.shift_right_logical(bi, 1)
            yv = plsc.bitcast(bi, jnp.float32)
            yv = yv * (1.5 - 0.5 * u * yv * yv)
            yv = yv * (1.5 - 0.5 * u * yv * yv)
            yv = yv * (1.5 - 0.5 * u * yv * yv)
            dis_v[pl.ds(i * 16, 16)] = yv
            return 0

        lax.fori_loop(0, RPT // 16, newton, 0)

        @pl.when(c == 0)
        def _():
            pltpu.async_copy(dis_v, dis_out.at[pl.ds(s * RPT, RPT)], sem_s)

        pltpu.make_async_copy(
            xw_hbm.at[pl.ds(s * RPT, RPT)], xw_v, sem_x).wait()

        @pl.when(c == 0)
        def _():
            pltpu.make_async_copy(
                dis_v, dis_out.at[pl.ds(s * RPT, RPT)], sem_s).wait()

        _sc_common_tail(c, s, wid, ei_hbm, sa_v, sb_v, rows_v, xw_v, 2,
                        st_v, dis_v, y_out.at[c], acc_sh, p_out, sem_g, sem_s)

    return k(xw1, ei)


@jax.jit
def _sc_layer2(xw2, dis, ei):
    @functools.partial(
        pl.kernel,
        mesh=_sc_mesh(),
        out_type=[
            jax.ShapeDtypeStruct((NC, PH, 4 * H), jnp.float32),
            jax.ShapeDtypeStruct((NC, NPAD, H), jnp.float32),
        ],
        scratch_types=[
            pltpu.VMEM((CH, CHUNK), jnp.int32),
            pltpu.VMEM((CH, CHUNK), jnp.int32),
            pltpu.VMEM((NBUF * CHUNK, H), jnp.float32),
            pltpu.VMEM((CHUNK, H), jnp.float32),
            pltpu.VMEM((CHUNK,), jnp.float32),
            pltpu.VMEM((CHUNK,), jnp.float32),
            pltpu.VMEM((RPT,), jnp.float32),
            pltpu.VMEM((PRT, 4 * H), jnp.float32),
            pltpu.VMEM((PRT, 4 * H), jnp.float32),
            pltpu.VMEM_SHARED((NPAD, H), jnp.float32),
            pltpu.SemaphoreType.DMA,
            pltpu.SemaphoreType.DMA,
            pltpu.SemaphoreType.DMA,
        ],
        compiler_params=_SC_PARAMS_NL,
    )
    def k(xw_hbm, dis_hbm, ei_hbm, p_out, y_out,
          sa_v, sb_v, rows_v, zr_v, ones_v, z1_v, dis_v, xw_v, st_v,
          acc_sh, sem_g, sem_s, sem_x):
        c = lax.axis_index("c")
        s = lax.axis_index("s")
        wid = c * NS + s
        _zero_buffers(zr_v, ones_v, z1_v)

        def zacc(i, _):
            pltpu.async_copy(
                zr_v, acc_sh.at[pl.ds(s * RPT + i * CHUNK, CHUNK)], sem_s)
            return 0

        lax.fori_loop(0, RPT // CHUNK, zacc, 0)
        pltpu.async_copy(dis_hbm.at[pl.ds(s * RPT, RPT)], dis_v, sem_x)
        pltpu.async_copy(xw_hbm.at[pl.ds(s * PRT, PRT)], xw_v, sem_x)

        def zdrain(i, _):
            pltpu.make_async_copy(
                zr_v, acc_sh.at[pl.ds(s * RPT, CHUNK)], sem_s).wait()
            return 0

        lax.fori_loop(0, RPT // CHUNK, zdrain, 0)
        pltpu.make_async_copy(
            dis_hbm.at[pl.ds(s * RPT, RPT)], dis_v, sem_x).wait()
        pltpu.make_async_copy(
            xw_hbm.at[pl.ds(s * PRT, PRT)], xw_v, sem_x).wait()
        _sc_common_tail(c, s, wid, ei_hbm, sa_v, sb_v, rows_v, xw_v, 8,
                        st_v, dis_v, y_out.at[c], acc_sh, p_out, sem_g, sem_s)

    return k(xw2, dis, ei)


def _tc_xw1(x, W1):
    def body(x_ref, w_ref, o_ref):
        o_ref[...] = jnp.dot(x_ref[...], w_ref[...],
                             preferred_element_type=jnp.float32)

    return pl.pallas_call(
        body,
        grid=(NPAD // RB,),
        in_specs=[
            pl.BlockSpec((RB, F), lambda i: (i, 0)),
            pl.BlockSpec((F, H), lambda i: (0, 0)),
        ],
        out_specs=pl.BlockSpec((RB, H), lambda i: (i, 0)),
        out_shape=jax.ShapeDtypeStruct((NPAD, H), jnp.float32),
    )(x, W1)


def _tc_mid(p1, b1b, W2b):
    def body(a0_ref, a1_ref, b_ref, w_ref, o_ref):
        h = jnp.maximum(a0_ref[0] + a1_ref[0] + b_ref[...], 0.0)
        o_ref[...] = jnp.dot(h, w_ref[...],
                             preferred_element_type=jnp.float32)

    return pl.pallas_call(
        body,
        grid=(PH // RBQ,),
        in_specs=[
            pl.BlockSpec((1, RBQ, 4 * H), lambda i: (0, i, 0)),
            pl.BlockSpec((1, RBQ, 4 * H), lambda i: (1, i, 0)),
            pl.BlockSpec((1, 4 * H), lambda i: (0, 0)),
            pl.BlockSpec((4 * H, 4 * H), lambda i: (0, 0)),
        ],
        out_specs=pl.BlockSpec((RBQ, 4 * H), lambda i: (i, 0)),
        out_shape=jax.ShapeDtypeStruct((PH, 4 * H), jnp.float32),
    )(p1, p1, b1b, W2b)


def _tc_out(p2, b2b, Wcb, bcb):
    def body(a0_ref, a1_ref, b_ref, wc_ref, bc_ref, o_ref):
        h = jnp.maximum(a0_ref[0] + a1_ref[0] + b_ref[...], 0.0)
        o_ref[...] = jnp.dot(
            h, wc_ref[...], preferred_element_type=jnp.float32) + bc_ref[...]

    return pl.pallas_call(
        body,
        grid=(PH // RBQ,),
        in_specs=[
            pl.BlockSpec((1, RBQ, 4 * H), lambda i: (0, i, 0)),
            pl.BlockSpec((1, RBQ, 4 * H), lambda i: (1, i, 0)),
            pl.BlockSpec((1, 4 * H), lambda i: (0, 0)),
            pl.BlockSpec((4 * H, O4), lambda i: (0, 0)),
            pl.BlockSpec((1, O4), lambda i: (0, 0)),
        ],
        out_specs=pl.BlockSpec((RBQ, O4), lambda i: (i, 0)),
        out_shape=jax.ShapeDtypeStruct((N // 4, O4), jnp.float32),
    )(p2, p2, b2b, Wcb, bcb)


def kernel(x, edge_index, W1, b1, W2, b2, Wc, bc):
    ei = edge_index.astype(jnp.int32)
    eye4 = jnp.eye(4, dtype=jnp.float32)
    W2b = jnp.kron(eye4, W2)
    Wcb = jnp.kron(eye4, Wc)
    b1b = jnp.tile(b1, 4).reshape(1, 4 * H)
    b2b = jnp.tile(b2, 4).reshape(1, 4 * H)
    bcb = jnp.tile(bc, 4).reshape(1, O4)

    xw1 = _tc_xw1(x, W1)
    p1, dis, _y1 = _sc_layer1(xw1, ei)
    xw2 = _tc_mid(p1, b1b, W2b)
    p2, _y2 = _sc_layer2(xw2, dis, ei)
    return _tc_out(p2, b2b, Wcb, bcb).reshape(N, O)
